# baseline (speedup 1.0000x reference)
import functools

import jax
import jax.numpy as jnp
from jax import lax
from jax.experimental import pallas as pl
from jax.experimental.pallas import tpu as pltpu
from jax.experimental.pallas import tpu_sc as plsc

NC = 2
NS = 16
NW = NC * NS

IDXW = 128
SUPER_IR = 8
SUB_IR = 2
SUB = SUB_IR * IDXW
NSLOT = SUPER_IR // SUB_IR
RETIRE_LAG = 2


def _make_gather(n_rows_total, d):
    rows_per_w = n_rows_total // NW
    ir_per_w = rows_per_w // IDXW
    n_super = ir_per_w // SUPER_IR
    mesh = plsc.VectorSubcoreMesh(core_axis_name="c", subcore_axis_name="s")

    @functools.partial(
        pl.kernel,
        mesh=mesh,
        out_type=jax.ShapeDtypeStruct((n_rows_total, d), jnp.float32),
        scratch_types=[
            pltpu.VMEM((SUPER_IR, IDXW), jnp.int32),
            pltpu.VMEM((SUPER_IR, IDXW), jnp.int32),
            pltpu.VMEM((SUB, d), jnp.float32),
            pltpu.VMEM((SUB, d), jnp.float32),
            pltpu.VMEM((SUB, d), jnp.float32),
            pltpu.VMEM((SUB, d), jnp.float32),
            pltpu.SemaphoreType.DMA,
            pltpu.SemaphoreType.DMA,
            pltpu.SemaphoreType.DMA,
            pltpu.SemaphoreType.DMA,
            pltpu.SemaphoreType.DMA,
            pltpu.SemaphoreType.DMA,
            pltpu.SemaphoreType.DMA,
            pltpu.SemaphoreType.DMA,
        ],
        compiler_params=pltpu.CompilerParams(use_tc_tiling_on_sc=False),
    )
    def gather_kernel(table_hbm, idx_hbm, out_hbm, idx_v0, idx_v1,
                      r0, r1, r2, r3, sg0, sg1, sg2, sg3,
                      so0, so1, so2, so3):
        wid = lax.axis_index("s") * NC + lax.axis_index("c")
        base_ir = wid * ir_per_w
        base_row = wid * rows_per_w
        idx_v = (idx_v0, idx_v1)
        rows = (r0, r1, r2, r3)
        sg = (sg0, sg1, sg2, sg3)
        so = (so0, so1, so2, so3)

        def idx_load(sc, ibuf):
            pltpu.sync_copy(
                idx_hbm.at[pl.ds(base_ir + sc * SUPER_IR, SUPER_IR)],
                idx_v[ibuf])

        def gather_descs(slot, ibuf):
            return [
                (table_hbm.at[idx_v[ibuf].at[slot * SUB_IR + j]],
                 rows[slot].at[pl.ds(j * IDXW, IDXW)], sg[slot])
                for j in range(SUB_IR)
            ]

        def out_ref(t):
            return out_hbm.at[pl.ds(base_row + t * SUB, SUB)]

        def step(t, k, do_free, do_retire):
            slot = k % NSLOT
            ibuf = (k // NSLOT) % 2
            if do_free:
                pltpu.make_async_copy(rows[slot], out_ref(t - NSLOT),
                                      so[slot]).wait()
            if k % NSLOT == 0:
                idx_load(t // NSLOT, ibuf)
            for a, b, s in gather_descs(slot, ibuf):
                pltpu.async_copy(a, b, s)
            if do_retire:
                k2 = (k - RETIRE_LAG) % (2 * NSLOT)
                slot2 = k2 % NSLOT
                ibuf2 = (k2 // NSLOT) % 2
                for a, b, s in gather_descs(slot2, ibuf2):
                    pltpu.make_async_copy(a, b, s).wait()
                pltpu.async_copy(rows[slot2], out_ref(t - RETIRE_LAG),
                                 so[slot2])

        for k in range(2 * NSLOT):
            step(k, k, do_free=(k >= NSLOT), do_retire=(k >= RETIRE_LAG))

        def body(p, carry):
            t0 = 2 * NSLOT + p * 2 * NSLOT
            for k in range(2 * NSLOT):
                step(t0 + k, k, do_free=True, do_retire=True)
            return carry

        lax.fori_loop(0, (n_super - 2) // 2, body, 0)

        if (n_super - 2) % 2:
            t0 = (n_super - 1) * NSLOT
            for k in range(NSLOT):
                step(t0 + k, k, do_free=True, do_retire=True)
        t_end = n_super * NSLOT
        for t in range(t_end - RETIRE_LAG, t_end):
            k2 = t % (2 * NSLOT)
            slot2 = k2 % NSLOT
            ibuf2 = (k2 // NSLOT) % 2
            for a, b, s in gather_descs(slot2, ibuf2):
                pltpu.make_async_copy(a, b, s).wait()
            pltpu.async_copy(rows[slot2], out_ref(t), so[slot2])
        for t in range(t_end - NSLOT, t_end):
            pltpu.make_async_copy(rows[t % NSLOT], out_ref(t),
                                  so[t % NSLOT]).wait()

    return gather_kernel


def kernel(x, table):
    b, h = x.shape
    v, d = table.shape
    ha = 32
    na, nb = b * ha, b * (h - ha)
    ia = x[:, :ha].reshape(na // IDXW, IDXW).astype(jnp.int32)
    ib = x[:, ha:].reshape(nb // IDXW, IDXW).astype(jnp.int32)
    oa = _make_gather(na, d)(table, ia).reshape(b, ha, d)
    ob = _make_gather(nb, d)(table, ib).reshape(b, h - ha, d)
    return jnp.concatenate([oa, ob], axis=1)


# speedup vs baseline: 1.0668x; 1.0668x over previous
import functools

import jax
import jax.numpy as jnp
from jax import lax
from jax.experimental import pallas as pl
from jax.experimental.pallas import tpu as pltpu
from jax.experimental.pallas import tpu_sc as plsc

NC = 2
NS = 16
NW = NC * NS

IDXW = 128
SUPER_IR = 8
SUB_IR = 2
SUB = SUB_IR * IDXW
NSLOT = SUPER_IR // SUB_IR
RETIRE_LAG = 2


def _make_gather(n_rows_total, d):
    rows_per_w = n_rows_total // NW
    ir_per_w = rows_per_w // IDXW
    n_super = ir_per_w // SUPER_IR
    mesh = plsc.VectorSubcoreMesh(core_axis_name="c", subcore_axis_name="s")

    @functools.partial(
        pl.kernel,
        mesh=mesh,
        out_type=jax.ShapeDtypeStruct((n_rows_total, d), jnp.float32),
        scratch_types=[
            pltpu.VMEM((SUPER_IR, IDXW), jnp.int32),
            pltpu.VMEM((SUPER_IR, IDXW), jnp.int32),
            pltpu.VMEM((SUB, d), jnp.float32),
            pltpu.VMEM((SUB, d), jnp.float32),
            pltpu.VMEM((SUB, d), jnp.float32),
            pltpu.VMEM((SUB, d), jnp.float32),
            pltpu.SemaphoreType.DMA,
            pltpu.SemaphoreType.DMA,
            pltpu.SemaphoreType.DMA,
            pltpu.SemaphoreType.DMA,
            pltpu.SemaphoreType.DMA,
            pltpu.SemaphoreType.DMA,
            pltpu.SemaphoreType.DMA,
            pltpu.SemaphoreType.DMA,
            pltpu.SemaphoreType.DMA,
            pltpu.SemaphoreType.DMA,
        ],
        compiler_params=pltpu.CompilerParams(use_tc_tiling_on_sc=False),
    )
    def gather_kernel(table_hbm, idx_hbm, out_hbm, idx_v0, idx_v1,
                      r0, r1, r2, r3, sg0, sg1, sg2, sg3,
                      so0, so1, so2, so3, si0, si1):
        wid = lax.axis_index("s") * NC + lax.axis_index("c")
        base_ir = wid * ir_per_w
        base_row = wid * rows_per_w
        idx_v = (idx_v0, idx_v1)
        rows = (r0, r1, r2, r3)
        sg = (sg0, sg1, sg2, sg3)
        so = (so0, so1, so2, so3)
        si = (si0, si1)

        def idx_desc(sc, ibuf):
            return (idx_hbm.at[pl.ds(base_ir + sc * SUPER_IR, SUPER_IR)],
                    idx_v[ibuf], si[ibuf])

        def gather_descs(slot, ibuf):
            return [
                (table_hbm.at[idx_v[ibuf].at[slot * SUB_IR + j]],
                 rows[slot].at[pl.ds(j * IDXW, IDXW)], sg[slot])
                for j in range(SUB_IR)
            ]

        def out_ref(t):
            return out_hbm.at[pl.ds(base_row + t * SUB, SUB)]

        def step(t, k, do_free, do_retire, prefetch):
            slot = k % NSLOT
            ibuf = (k // NSLOT) % 2
            if do_free:
                pltpu.make_async_copy(rows[slot], out_ref(t - NSLOT),
                                      so[slot]).wait()
            if k % NSLOT == 0:
                a, b, s = idx_desc(t // NSLOT, ibuf)
                pltpu.make_async_copy(a, b, s).wait()
            for a, b, s in gather_descs(slot, ibuf):
                pltpu.async_copy(a, b, s)
            if do_retire:
                k2 = (k - RETIRE_LAG) % (2 * NSLOT)
                slot2 = k2 % NSLOT
                ibuf2 = (k2 // NSLOT) % 2
                for a, b, s in gather_descs(slot2, ibuf2):
                    pltpu.make_async_copy(a, b, s).wait()
                pltpu.async_copy(rows[slot2], out_ref(t - RETIRE_LAG),
                                 so[slot2])
            if prefetch and k % NSLOT == 1:
                sc1 = t // NSLOT + 1
                other = 1 - ibuf

                def fire():
                    a, b, s = idx_desc(sc1, other)
                    pltpu.async_copy(a, b, s)

                if isinstance(t, int):
                    if sc1 < n_super:
                        fire()
                else:
                    pl.when(sc1 < n_super)(fire)

        a, b, s = idx_desc(0, 0)
        pltpu.async_copy(a, b, s)
        a, b, s = idx_desc(1, 1)
        pltpu.async_copy(a, b, s)
        for k in range(2 * NSLOT):
            step(k, k, do_free=(k >= NSLOT), do_retire=(k >= RETIRE_LAG),
                 prefetch=(k >= NSLOT))

        def body(p, carry):
            t0 = 2 * NSLOT + p * 2 * NSLOT
            for k in range(2 * NSLOT):
                step(t0 + k, k, do_free=True, do_retire=True, prefetch=True)
            return carry

        lax.fori_loop(0, (n_super - 2) // 2, body, 0)

        if (n_super - 2) % 2:
            t0 = (n_super - 1) * NSLOT
            for k in range(NSLOT):
                step(t0 + k, k, do_free=True, do_retire=True,
                     prefetch=False)
        t_end = n_super * NSLOT
        for t in range(t_end - RETIRE_LAG, t_end):
            k2 = t % (2 * NSLOT)
            slot2 = k2 % NSLOT
            ibuf2 = (k2 // NSLOT) % 2
            for a, b, s in gather_descs(slot2, ibuf2):
                pltpu.make_async_copy(a, b, s).wait()
            pltpu.async_copy(rows[slot2], out_ref(t), so[slot2])
        for t in range(t_end - NSLOT, t_end):
            pltpu.make_async_copy(rows[t % NSLOT], out_ref(t),
                                  so[t % NSLOT]).wait()

    return gather_kernel


def kernel(x, table):
    b, h = x.shape
    v, d = table.shape
    n = b * h
    idx2d = x.reshape(n // IDXW, IDXW).astype(jnp.int32)
    out = _make_gather(n, d)(table, idx2d)
    return out.reshape(b, h, d)


# speedup vs baseline: 1.0668x; 1.0000x over previous
import functools

import jax
import jax.numpy as jnp
from jax import lax
from jax.experimental import pallas as pl
from jax.experimental.pallas import tpu as pltpu
from jax.experimental.pallas import tpu_sc as plsc

NC = 2
NS = 16
NW = NC * NS

IDXW = 128
SUPER_IR = 8
SUB_IR = 2
SUB = SUB_IR * IDXW
NSLOT = SUPER_IR // SUB_IR
RETIRE_LAG = 3


def _make_gather(n_rows_total, d):
    rows_per_w = n_rows_total // NW
    ir_per_w = rows_per_w // IDXW
    n_super = ir_per_w // SUPER_IR
    mesh = plsc.VectorSubcoreMesh(core_axis_name="c", subcore_axis_name="s")

    @functools.partial(
        pl.kernel,
        mesh=mesh,
        out_type=jax.ShapeDtypeStruct((n_rows_total, d), jnp.float32),
        scratch_types=[
            pltpu.VMEM((SUPER_IR, IDXW), jnp.int32),
            pltpu.VMEM((SUPER_IR, IDXW), jnp.int32),
            pltpu.VMEM((SUB, d), jnp.float32),
            pltpu.VMEM((SUB, d), jnp.float32),
            pltpu.VMEM((SUB, d), jnp.float32),
            pltpu.VMEM((SUB, d), jnp.float32),
            pltpu.SemaphoreType.DMA,
            pltpu.SemaphoreType.DMA,
            pltpu.SemaphoreType.DMA,
            pltpu.SemaphoreType.DMA,
            pltpu.SemaphoreType.DMA,
            pltpu.SemaphoreType.DMA,
            pltpu.SemaphoreType.DMA,
            pltpu.SemaphoreType.DMA,
            pltpu.SemaphoreType.DMA,
            pltpu.SemaphoreType.DMA,
        ],
        compiler_params=pltpu.CompilerParams(use_tc_tiling_on_sc=False),
    )
    def gather_kernel(table_hbm, idx_hbm, out_hbm, idx_v0, idx_v1,
                      r0, r1, r2, r3, sg0, sg1, sg2, sg3,
                      so0, so1, so2, so3, si0, si1):
        wid = lax.axis_index("s") * NC + lax.axis_index("c")
        base_ir = wid * ir_per_w
        base_row = wid * rows_per_w
        idx_v = (idx_v0, idx_v1)
        rows = (r0, r1, r2, r3)
        sg = (sg0, sg1, sg2, sg3)
        so = (so0, so1, so2, so3)
        si = (si0, si1)

        def idx_desc(sc, ibuf):
            return (idx_hbm.at[pl.ds(base_ir + sc * SUPER_IR, SUPER_IR)],
                    idx_v[ibuf], si[ibuf])

        def gather_descs(slot, ibuf):
            return [
                (table_hbm.at[idx_v[ibuf].at[slot * SUB_IR + j]],
                 rows[slot].at[pl.ds(j * IDXW, IDXW)], sg[slot])
                for j in range(SUB_IR)
            ]

        def out_ref(t):
            return out_hbm.at[pl.ds(base_row + t * SUB, SUB)]

        def step(t, k, do_free, do_retire, prefetch):
            slot = k % NSLOT
            ibuf = (k // NSLOT) % 2
            if do_free:
                pltpu.make_async_copy(rows[slot], out_ref(t - NSLOT),
                                      so[slot]).wait()
            if k % NSLOT == 0:
                a, b, s = idx_desc(t // NSLOT, ibuf)
                pltpu.make_async_copy(a, b, s).wait()
            for a, b, s in gather_descs(slot, ibuf):
                pltpu.async_copy(a, b, s)
            if do_retire:
                k2 = (k - RETIRE_LAG) % (2 * NSLOT)
                slot2 = k2 % NSLOT
                ibuf2 = (k2 // NSLOT) % 2
                for a, b, s in gather_descs(slot2, ibuf2):
                    pltpu.make_async_copy(a, b, s).wait()
                pltpu.async_copy(rows[slot2], out_ref(t - RETIRE_LAG),
                                 so[slot2])
            if prefetch and k % NSLOT == 1:
                sc1 = t // NSLOT + 1
                other = 1 - ibuf

                def fire():
                    a, b, s = idx_desc(sc1, other)
                    pltpu.async_copy(a, b, s)

                if isinstance(t, int):
                    if sc1 < n_super:
                        fire()
                else:
                    pl.when(sc1 < n_super)(fire)

        a, b, s = idx_desc(0, 0)
        pltpu.async_copy(a, b, s)
        a, b, s = idx_desc(1, 1)
        pltpu.async_copy(a, b, s)
        for k in range(2 * NSLOT):
            step(k, k, do_free=(k >= NSLOT), do_retire=(k >= RETIRE_LAG),
                 prefetch=(k >= NSLOT))

        def body(p, carry):
            t0 = 2 * NSLOT + p * 2 * NSLOT
            for k in range(2 * NSLOT):
                step(t0 + k, k, do_free=True, do_retire=True, prefetch=True)
            return carry

        lax.fori_loop(0, (n_super - 2) // 2, body, 0)

        if (n_super - 2) % 2:
            t0 = (n_super - 1) * NSLOT
            for k in range(NSLOT):
                step(t0 + k, k, do_free=True, do_retire=True,
                     prefetch=False)
        t_end = n_super * NSLOT
        for t in range(t_end - RETIRE_LAG, t_end):
            k2 = t % (2 * NSLOT)
            slot2 = k2 % NSLOT
            ibuf2 = (k2 // NSLOT) % 2
            for a, b, s in gather_descs(slot2, ibuf2):
                pltpu.make_async_copy(a, b, s).wait()
            pltpu.async_copy(rows[slot2], out_ref(t), so[slot2])
        for t in range(t_end - NSLOT, t_end):
            pltpu.make_async_copy(rows[t % NSLOT], out_ref(t),
                                  so[t % NSLOT]).wait()

    return gather_kernel


def kernel(x, table):
    b, h = x.shape
    v, d = table.shape
    n = b * h
    idx2d = x.reshape(n // IDXW, IDXW).astype(jnp.int32)
    out = _make_gather(n, d)(table, idx2d)
    return out.reshape(b, h, d)
